# baseline (speedup 1.0000x reference)
import functools
import math

import jax
import jax.numpy as jnp
from jax import lax
from jax.experimental import pallas as pl
from jax.experimental.pallas import tpu as pltpu
from jax.experimental.pallas import tpu_sc as plsc

VOCAB = 1_000_000
DIM = 64
BATCH = 4096
SEQ = 200
NC, NS = 2, 16
NW = NC * NS
BBLK = BATCH // NW
NBUF = 4
SCALE = math.sqrt(DIM)

_mesh = plsc.VectorSubcoreMesh(
    core_axis_name="c", subcore_axis_name="s", num_cores=NC, num_subcores=NS
)


@functools.partial(
    pl.kernel,
    out_type=jax.ShapeDtypeStruct((SEQ, DIM // 8, NW, 8 * BBLK), jnp.float32),
    mesh=_mesh,
    compiler_params=pltpu.CompilerParams(
        use_tc_tiling_on_sc=False, needs_layout_passes=False
    ),
    scratch_types=[
        pltpu.VMEM((SEQ, BBLK), jnp.int32),
        pltpu.VMEM((NBUF, BBLK, DIM), jnp.float32),
        pltpu.VMEM((NBUF, DIM // 8, 8 * BBLK), jnp.float32),
        [pltpu.SemaphoreType.DMA] * NBUF,
        [pltpu.SemaphoreType.DMA] * NBUF,
    ],
)
def _embed(ids_hbm, tab_hbm, out_hbm, idx_v, gbuf, obuf, gsem, osem):
    wid = lax.axis_index("s") * NC + lax.axis_index("c")
    pltpu.sync_copy(ids_hbm.at[:, pl.ds(wid * BBLK, BBLK)], idx_v)

    lane = lax.broadcasted_iota(jnp.int32, (16,), 0)
    row_idx = [lane + (g * 16) for g in range(BBLK // 16)]

    for b in range(NBUF):
        pltpu.async_copy(tab_hbm.at[idx_v.at[b]], gbuf.at[b], gsem[b])

    def transpose_scale(b):
        def d_body(d, carry):
            dh = lax.shift_right_logical(d, 3)
            dl = lax.bitwise_and(d, 7)
            col = jnp.full((16,), d, jnp.int32)
            for g in range(BBLK // 16):
                vals = plsc.load_gather(gbuf.at[b], [row_idx[g], col])
                obuf[b, dh, pl.ds(dl * BBLK + g * 16, 16)] = vals * SCALE
            return carry

        lax.fori_loop(0, DIM, d_body, 0)

    def chunk_body(t, carry):
        for b in range(NBUF):
            s = t * NBUF + b
            pltpu.make_async_copy(
                tab_hbm.at[pl.ds(0, BBLK)], gbuf.at[b], gsem[b]
            ).wait()

            @pl.when(t > 0)
            def _():
                pltpu.make_async_copy(
                    obuf.at[b], out_hbm.at[0, :, 0], osem[b]
                ).wait()

            transpose_scale(b)

            pltpu.async_copy(obuf.at[b], out_hbm.at[s, :, wid], osem[b])

            @pl.when(t < SEQ // NBUF - 1)
            def _():
                pltpu.async_copy(
                    tab_hbm.at[idx_v.at[s + NBUF]], gbuf.at[b], gsem[b]
                )
        return carry

    lax.fori_loop(0, SEQ // NBUF, chunk_body, 0)

    for b in range(NBUF):
        pltpu.make_async_copy(obuf.at[b], out_hbm.at[0, :, 0], osem[b]).wait()


def kernel(token_ids_batch, embeddings_table):
    ids_t = token_ids_batch.astype(jnp.int32).T
    out5 = _embed(ids_t, embeddings_table)
    out = (
        out5.reshape(SEQ, DIM // 8, NW, 8, BBLK)
        .transpose(2, 4, 0, 1, 3)
        .reshape(BATCH, SEQ, DIM)
    )
    return out


# speedup vs baseline: 1.1327x; 1.1327x over previous
import functools
import math

import jax
import jax.numpy as jnp
from jax import lax
from jax.experimental import pallas as pl
from jax.experimental.pallas import tpu as pltpu
from jax.experimental.pallas import tpu_sc as plsc

VOCAB = 1_000_000
DIM = 64
BATCH = 4096
SEQ = 200
NC, NS = 2, 16
NW = NC * NS
BBLK = BATCH // NW
NBUF = 4
SCALE = math.sqrt(DIM)

_mesh = plsc.VectorSubcoreMesh(
    core_axis_name="c", subcore_axis_name="s", num_cores=NC, num_subcores=NS
)


@functools.partial(
    pl.kernel,
    out_type=jax.ShapeDtypeStruct((SEQ, DIM // 8, NW, 8 * BBLK), jnp.float32),
    mesh=_mesh,
    compiler_params=pltpu.CompilerParams(
        use_tc_tiling_on_sc=False, needs_layout_passes=False
    ),
    scratch_types=[
        pltpu.VMEM((SEQ, BBLK), jnp.int32),
        pltpu.VMEM((NBUF, BBLK, DIM), jnp.float32),
        pltpu.VMEM((NBUF, DIM // 8, 8 * BBLK), jnp.float32),
        [pltpu.SemaphoreType.DMA] * NBUF,
        [pltpu.SemaphoreType.DMA] * NBUF,
    ],
)
def _embed(ids_hbm, tab_hbm, out_hbm, idx_v, gbuf, obuf, gsem, osem):
    wid = lax.axis_index("s") * NC + lax.axis_index("c")
    pltpu.sync_copy(ids_hbm.at[:, pl.ds(wid * BBLK, BBLK)], idx_v)

    lane = lax.broadcasted_iota(jnp.int32, (16,), 0)
    dh_idx = [lax.shift_right_logical(lane, 3) + 2 * g for g in range(DIM // 16)]
    in_idx = [lax.bitwise_and(lane, 7) * BBLK for g in range(DIM // 16)]

    for b in range(NBUF):
        pltpu.async_copy(tab_hbm.at[idx_v.at[b]], gbuf.at[b], gsem[b])

    def transpose_scale(b):
        def t_body(tok, carry):
            for g in range(DIM // 16):
                vals = gbuf[b, tok, pl.ds(g * 16, 16)]
                plsc.store_scatter(
                    obuf.at[b], [dh_idx[g], in_idx[g] + tok], vals * SCALE
                )
            return carry

        lax.fori_loop(0, BBLK, t_body, 0, unroll=8)

    def chunk_body(t, carry):
        for b in range(NBUF):
            s = t * NBUF + b
            pltpu.make_async_copy(
                tab_hbm.at[pl.ds(0, BBLK)], gbuf.at[b], gsem[b]
            ).wait()

            @pl.when(t > 0)
            def _():
                pltpu.make_async_copy(
                    obuf.at[b], out_hbm.at[0, :, 0], osem[b]
                ).wait()

            transpose_scale(b)

            pltpu.async_copy(obuf.at[b], out_hbm.at[s, :, wid], osem[b])

            @pl.when(t < SEQ // NBUF - 1)
            def _():
                pltpu.async_copy(
                    tab_hbm.at[idx_v.at[s + NBUF]], gbuf.at[b], gsem[b]
                )
        return carry

    lax.fori_loop(0, SEQ // NBUF, chunk_body, 0)

    for b in range(NBUF):
        pltpu.make_async_copy(obuf.at[b], out_hbm.at[0, :, 0], osem[b]).wait()


def kernel(token_ids_batch, embeddings_table):
    ids_t = token_ids_batch.astype(jnp.int32).T
    out5 = _embed(ids_t, embeddings_table)
    out = (
        out5.reshape(SEQ, DIM // 8, NW, 8, BBLK)
        .transpose(2, 4, 0, 1, 3)
        .reshape(BATCH, SEQ, DIM)
    )
    return out


# speedup vs baseline: 1.6202x; 1.4305x over previous
import functools
import math

import jax
import jax.numpy as jnp
from jax import lax
from jax.experimental import pallas as pl
from jax.experimental.pallas import tpu as pltpu
from jax.experimental.pallas import tpu_sc as plsc

VOCAB = 1_000_000
DIM = 64
BATCH = 4096
SEQ = 200
NC, NS = 2, 16
NW = NC * NS
BBLK = BATCH // NW
NBUF = 4
SCALE = math.sqrt(DIM)

_mesh = plsc.VectorSubcoreMesh(
    core_axis_name="c", subcore_axis_name="s", num_cores=NC, num_subcores=NS
)


PADB = BBLK + 9


@functools.partial(
    pl.kernel,
    out_type=jax.ShapeDtypeStruct((SEQ, DIM // 8, NW, 8, BBLK), jnp.float32),
    mesh=_mesh,
    compiler_params=pltpu.CompilerParams(
        use_tc_tiling_on_sc=False, needs_layout_passes=False
    ),
    scratch_types=[
        pltpu.VMEM((SEQ, BBLK), jnp.int32),
        pltpu.VMEM((NBUF, BBLK, DIM), jnp.float32),
        pltpu.VMEM((NBUF, DIM // 8, 8, PADB), jnp.float32),
        [pltpu.SemaphoreType.DMA] * NBUF,
        [pltpu.SemaphoreType.DMA] * NBUF,
    ],
)
def _embed(ids_hbm, tab_hbm, out_hbm, idx_v, gbuf, obuf, gsem, osem):
    wid = lax.axis_index("s") * NC + lax.axis_index("c")
    pltpu.sync_copy(ids_hbm.at[:, pl.ds(wid * BBLK, BBLK)], idx_v)

    lane = lax.broadcasted_iota(jnp.int32, (16,), 0)
    dh_idx = [lax.shift_right_logical(lane, 3) + 2 * g for g in range(DIM // 16)]
    dl_idx = lax.bitwise_and(lane, 7)

    for b in range(NBUF):
        pltpu.async_copy(tab_hbm.at[idx_v.at[b]], gbuf.at[b], gsem[b])

    def transpose_scale(b):
        def t_body(tok, carry):
            tokv = jnp.full((16,), tok, jnp.int32)
            for g in range(DIM // 16):
                vals = gbuf[b, tok, pl.ds(g * 16, 16)]
                plsc.store_scatter(
                    obuf.at[b], [dh_idx[g], dl_idx, tokv], vals * SCALE
                )
            return carry

        lax.fori_loop(0, BBLK, t_body, 0, unroll=8)

    def chunk_body(t, carry):
        for b in range(NBUF):
            s = t * NBUF + b
            pltpu.make_async_copy(
                tab_hbm.at[pl.ds(0, BBLK)], gbuf.at[b], gsem[b]
            ).wait()

            @pl.when(t > 0)
            def _():
                pltpu.make_async_copy(
                    obuf.at[b, :, :, pl.ds(0, BBLK)], out_hbm.at[0, :, 0], osem[b]
                ).wait()

            transpose_scale(b)

            pltpu.async_copy(
                obuf.at[b, :, :, pl.ds(0, BBLK)], out_hbm.at[s, :, wid], osem[b]
            )

            @pl.when(t < SEQ // NBUF - 1)
            def _():
                pltpu.async_copy(
                    tab_hbm.at[idx_v.at[s + NBUF]], gbuf.at[b], gsem[b]
                )
        return carry

    lax.fori_loop(0, SEQ // NBUF, chunk_body, 0)

    for b in range(NBUF):
        pltpu.make_async_copy(
            obuf.at[b, :, :, pl.ds(0, BBLK)], out_hbm.at[0, :, 0], osem[b]
        ).wait()


def kernel(token_ids_batch, embeddings_table):
    ids_t = token_ids_batch.astype(jnp.int32).T
    out5 = _embed(ids_t, embeddings_table)
    out = (
        out5.reshape(SEQ, DIM // 8, NW, 8, BBLK)
        .transpose(2, 4, 0, 1, 3)
        .reshape(BATCH, SEQ, DIM)
    )
    return out
